# Initial kernel scaffold; baseline (speedup 1.0000x reference)
import jax
import jax.numpy as jnp
from jax.experimental import pallas as pl


def kernel(x, input_idx, input_weights, process_indices, input_neurons, process_neurons):
    raise NotImplementedError("write your pallas kernel here")


# speedup vs baseline: 3.6884x; 3.6884x over previous
import functools

import jax
import jax.numpy as jnp
from jax import lax
from jax.experimental import pallas as pl

B, S, D, R, N_INPUT, N_PROCESS, K = 4, 2048, 2048, 256, 8, 32, 8
T_BLK = 512


def _fused_kernel(x_ref, w_ref, pidx_ref, wstk_ref, p_ref, out_ref):
    x_blk = x_ref[...]
    w_blk = w_ref[...]
    pidx = pidx_ref[...]
    wstk = wstk_ref[...]
    p = p_ref[...]

    big = jnp.dot(x_blk, wstk, preferred_element_type=jnp.float32)
    h0 = jnp.zeros((x_blk.shape[0], R), dtype=jnp.float32)
    for n in range(N_INPUT):
        h0 = h0 + big[:, n * R:(n + 1) * R] * w_blk[:, n:n + 1]

    vnorm = jnp.sum(p * p, axis=1, keepdims=True) + 1e-8
    vn = p * lax.rsqrt(vnorm)
    gn = lax.dot_general(vn, vn, (((1,), (1,)), ((), ())),
                         preferred_element_type=jnp.float32)

    d = lax.dot_general(h0, vn, (((1,), (1,)), ((), ())),
                        preferred_element_type=jnp.float32)
    coeff = jnp.zeros_like(d)
    ids = lax.broadcasted_iota(jnp.int32, (1, N_PROCESS), 1)
    for k in range(K):
        onehot = (pidx[:, k:k + 1] == ids).astype(jnp.float32)
        c2 = 2.0 * jnp.sum(onehot * d, axis=1, keepdims=True)
        g = jnp.dot(onehot, gn, preferred_element_type=jnp.float32)
        d = d - c2 * g
        coeff = coeff + c2 * onehot

    out_ref[...] = h0 - jnp.dot(coeff, vn, preferred_element_type=jnp.float32)


@jax.jit
def kernel(x, input_idx, input_weights, process_indices, input_neurons, process_neurons):
    del input_idx
    T = B * S
    xf = x.reshape(T, D)
    wf = input_weights.reshape(T, N_INPUT)
    pidxf = process_indices.reshape(T, K).astype(jnp.int32)
    wstk = input_neurons.transpose(1, 0, 2).reshape(D, N_INPUT * R)

    grid = (T // T_BLK,)
    out = pl.pallas_call(
        _fused_kernel,
        grid=grid,
        in_specs=[
            pl.BlockSpec((T_BLK, D), lambda i: (i, 0)),
            pl.BlockSpec((T_BLK, N_INPUT), lambda i: (i, 0)),
            pl.BlockSpec((T_BLK, K), lambda i: (i, 0)),
            pl.BlockSpec((D, N_INPUT * R), lambda i: (0, 0)),
            pl.BlockSpec((N_PROCESS, R), lambda i: (0, 0)),
        ],
        out_specs=pl.BlockSpec((T_BLK, R), lambda i: (i, 0)),
        out_shape=jax.ShapeDtypeStruct((T, R), jnp.float32),
    )(xf, wf, pidxf, wstk, process_neurons)
    return out.reshape(B, S, R)
